# Initial kernel scaffold; baseline (speedup 1.0000x reference)
import functools

import jax
import jax.numpy as jnp
from jax import lax
from jax.experimental import pallas as pl
from jax.experimental.pallas import tpu as pltpu
from jax.experimental.pallas import tpu_sc as plsc

N = 320000
D = 128
M = 10000

NW = 32
WSEG = 32
NWIN = -(-M // WSEG)
M_PAD = NWIN * WSEG
T1 = 256
G = 256
NLANE = 16
NEG = jnp.float32(-jnp.inf)

_MESH = plsc.VectorSubcoreMesh(core_axis_name="c", subcore_axis_name="s")


def _worker_id():
    return lax.axis_index("s") * 2 + lax.axis_index("c")


@functools.partial(
    pl.kernel,
    out_type=(
        jax.ShapeDtypeStruct((M_PAD, D), jnp.float32),
        jax.ShapeDtypeStruct((M_PAD, D), jnp.float32),
        jax.ShapeDtypeStruct((M_PAD, D), jnp.float32),
    ),
    mesh=_MESH,
    scratch_types=[
        pltpu.VMEM((T1, D), jnp.float32),
        pltpu.VMEM((WSEG, D), jnp.float32),
        pltpu.VMEM((WSEG, D), jnp.float32),
        pltpu.VMEM((WSEG, D), jnp.float32),
        pltpu.SMEM((T1,), jnp.int32),
        pltpu.SMEM((8,), jnp.int32),
        pltpu.SMEM((2,), jnp.int32),
        pltpu.SMEM((WSEG,), jnp.float32),
    ],
)
def _seg_reduce(feat_hbm, ids_hbm, params_hbm, sum_hbm, max_hbm, cnt_hbm,
                feat_v, accs_v, accm_v, cstg_v, ids_s, par_s, st_s, cnt_s):
    wid = _worker_id()
    pltpu.sync_copy(params_hbm.at[wid], par_s)
    w0 = par_s[0]
    w1 = par_s[1]
    rlo = par_s[2]
    rhi = par_s[3]

    def reinit():
        zero = jnp.zeros((NLANE,), jnp.float32)
        neg = jnp.full((NLANE,), NEG)

        @pl.loop(0, WSEG)
        def _(s):
            for j in range(D // NLANE):
                accs_v[s, pl.ds(j * NLANE, NLANE)] = zero
                accm_v[s, pl.ds(j * NLANE, NLANE)] = neg
            cnt_s[s] = jnp.float32(0.0)

    def flush(w):
        @pl.loop(0, WSEG)
        def _(s):
            cvec = jnp.full((NLANE,), cnt_s[s])
            for j in range(D // NLANE):
                cstg_v[s, pl.ds(j * NLANE, NLANE)] = cvec

        base = w * WSEG
        pltpu.sync_copy(accs_v, sum_hbm.at[pl.ds(base, WSEG), :])
        pltpu.sync_copy(accm_v, max_hbm.at[pl.ds(base, WSEG), :])
        pltpu.sync_copy(cstg_v, cnt_hbm.at[pl.ds(base, WSEG), :])

    @pl.when(w1 > w0)
    def _():
        reinit()
        st_s[0] = w0

        t0 = rlo // T1
        t1 = (rhi + T1 - 1) // T1

        @pl.loop(t0, t1)
        def _(t):
            base = t * T1
            pltpu.sync_copy(ids_hbm.at[pl.ds(base, T1)], ids_s)
            pltpu.sync_copy(feat_hbm.at[pl.ds(base, T1), :], feat_v)
            ilo = jnp.maximum(rlo - base, 0)
            ihi = jnp.minimum(rhi - base, T1)

            @pl.loop(ilo, ihi)
            def _(i):
                m = ids_s[i]
                wt = m // WSEG
                cw = st_s[0]

                @pl.when(wt > cw)
                def _():
                    flush(cw)
                    reinit()

                    @pl.loop(cw + 1, wt)
                    def _(w):
                        flush(w)

                    st_s[0] = wt

                ls = m - st_s[0] * WSEG
                for j in range(D // NLANE):
                    sl = pl.ds(j * NLANE, NLANE)
                    v = feat_v[i, sl]
                    accm_v[ls, sl] = jnp.maximum(accm_v[ls, sl], v)
                    accs_v[ls, sl] = accs_v[ls, sl] + v
                cnt_s[ls] = cnt_s[ls] + jnp.float32(1.0)

        fc = st_s[0]
        flush(fc)
        reinit()

        @pl.loop(fc + 1, w1)
        def _(w):
            flush(w)


def _bn_gelu_body(sum_ref, max_ref, cnt_ref, gamma_ref, beta_ref, out_ref):
    c = cnt_ref[...]
    nonempty = c > 0.0
    s = jnp.where(nonempty,
                  jnp.where(nonempty, max_ref[...], 0.0)
                  + sum_ref[...] / jnp.maximum(c, 1.0),
                  0.0)
    inv_m = jnp.float32(1.0 / M)
    mu = jnp.sum(s, axis=0, keepdims=True) * inv_m
    valid = lax.broadcasted_iota(jnp.int32, (M_PAD, D), 0) < M
    d = jnp.where(valid, s - mu, 0.0)
    var = jnp.sum(d * d, axis=0, keepdims=True) * inv_m
    x = (s - mu) * lax.rsqrt(var + 1e-5) * gamma_ref[...] + beta_ref[...]
    out_ref[...] = 0.5 * x * (1.0 + lax.erf(x * jnp.float32(0.7071067811865476)))


@functools.partial(
    pl.kernel,
    out_type=jax.ShapeDtypeStruct((N, D), jnp.float32),
    mesh=_MESH,
    scratch_types=[
        pltpu.VMEM((G // 128, 128), jnp.int32),
        pltpu.VMEM((G, D), jnp.float32),
        pltpu.VMEM((G, D), jnp.float32),
    ],
)
def _gather_add(feat_hbm, ids2_hbm, table_hbm, out_hbm, idx_v, feat_v, rows_v):
    wid = _worker_id()
    ntiles = N // G

    @pl.loop(wid, ntiles, step=NW)
    def _(t):
        base = t * G
        pltpu.sync_copy(ids2_hbm.at[pl.ds(t * (G // 128), G // 128), :], idx_v)
        pltpu.sync_copy(feat_hbm.at[pl.ds(base, G), :], feat_v)
        for j in range(G // 128):
            pltpu.sync_copy(table_hbm.at[idx_v.at[j]],
                            rows_v.at[pl.ds(j * 128, 128), :])

        @pl.loop(0, G)
        def _(i):
            for j in range(D // NLANE):
                sl = pl.ds(j * NLANE, NLANE)
                rows_v[i, sl] = rows_v[i, sl] + feat_v[i, sl]

        pltpu.sync_copy(rows_v, out_hbm.at[pl.ds(base, G), :])


def kernel(feat, segment_ids, bn_gamma, bn_beta):
    ids = segment_ids.astype(jnp.int32)

    win_row_start = jnp.searchsorted(
        ids, jnp.arange(NWIN + 1, dtype=jnp.int32) * WSEG).astype(jnp.int32)
    target = jnp.arange(NW + 1, dtype=jnp.int32) * (N // NW)
    wk = jnp.searchsorted(win_row_start, target, side="left").astype(jnp.int32)
    wk = wk.at[0].set(0).at[-1].set(NWIN)
    rows = win_row_start[wk]
    zeros = jnp.zeros((NW,), jnp.int32)
    params = jnp.stack(
        [wk[:-1], wk[1:], rows[:-1], rows[1:], zeros, zeros, zeros, zeros],
        axis=1)

    seg_sum, seg_max, seg_cnt = _seg_reduce(feat, ids, params)

    s_act = pl.pallas_call(
        _bn_gelu_body,
        out_shape=jax.ShapeDtypeStruct((M_PAD, D), jnp.float32),
    )(seg_sum, seg_max, seg_cnt,
      bn_gamma.reshape(1, D).astype(jnp.float32),
      bn_beta.reshape(1, D).astype(jnp.float32))

    ids2 = ids.reshape(N // 128, 128)
    return _gather_add(feat, ids2, s_act)


# speedup vs baseline: 1.3127x; 1.3127x over previous
import functools

import jax
import jax.numpy as jnp
from jax import lax
from jax.experimental import pallas as pl
from jax.experimental.pallas import tpu as pltpu
from jax.experimental.pallas import tpu_sc as plsc

N = 320000
D = 128
M = 10000

NW = 32
WSEG = 32
NWIN = -(-M // WSEG)
M_PAD = NWIN * WSEG
T1 = 256
G = 256
NLANE = 16
NEG = float("-inf")

_MESH = plsc.VectorSubcoreMesh(core_axis_name="c", subcore_axis_name="s")


def _worker_id():
    return lax.axis_index("s") * 2 + lax.axis_index("c")


@functools.partial(
    pl.kernel,
    out_type=(
        jax.ShapeDtypeStruct((M_PAD, D), jnp.float32),
        jax.ShapeDtypeStruct((M_PAD, D), jnp.float32),
        jax.ShapeDtypeStruct((M_PAD, D), jnp.float32),
    ),
    mesh=_MESH,
    scratch_types=[
        pltpu.VMEM((T1, D), jnp.float32),
        pltpu.VMEM((WSEG, D), jnp.float32),
        pltpu.VMEM((WSEG, D), jnp.float32),
        pltpu.VMEM((WSEG, D), jnp.float32),
        pltpu.VMEM((T1,), jnp.int32),
        pltpu.VMEM((NLANE,), jnp.int32),
        pltpu.SMEM((2,), jnp.int32),
        pltpu.SMEM((WSEG,), jnp.float32),
    ],
)
def _seg_reduce(feat_hbm, ids_hbm, params_hbm, sum_hbm, max_hbm, cnt_hbm,
                feat_v, accs_v, accm_v, cstg_v, ids_v, par_v, st_s, cnt_s):
    wid = _worker_id()
    pltpu.sync_copy(params_hbm.at[wid], par_v)
    pv = par_v[...]
    w0 = pv[0]
    w1 = pv[1]
    rlo = pv[2]
    rhi = pv[3]

    def reinit():
        zero = jnp.zeros((NLANE,), jnp.float32)
        neg = jnp.full((NLANE,), NEG, jnp.float32)

        @pl.loop(0, WSEG)
        def _(s):
            for j in range(D // NLANE):
                accs_v[s, pl.ds(j * NLANE, NLANE)] = zero
                accm_v[s, pl.ds(j * NLANE, NLANE)] = neg
            cnt_s[s] = jnp.float32(0.0)

    def flush(w):
        @pl.loop(0, WSEG)
        def _(s):
            cvec = jnp.full((NLANE,), cnt_s[s])
            for j in range(D // NLANE):
                cstg_v[s, pl.ds(j * NLANE, NLANE)] = cvec

        base = w * WSEG
        pltpu.sync_copy(accs_v, sum_hbm.at[pl.ds(base, WSEG), :])
        pltpu.sync_copy(accm_v, max_hbm.at[pl.ds(base, WSEG), :])
        pltpu.sync_copy(cstg_v, cnt_hbm.at[pl.ds(base, WSEG), :])

    def cross(m):
        wt = m // WSEG
        cw = st_s[0]

        @pl.when(wt > cw)
        def _():
            flush(cw)
            reinit()

            @pl.loop(cw + 1, wt)
            def _(w):
                flush(w)

            st_s[0] = wt

    @pl.when(w1 > w0)
    def _():
        reinit()
        st_s[0] = w0

        t0 = rlo // T1
        t1 = (rhi + T1 - 1) // T1

        @pl.loop(t0, t1)
        def _(t):
            base = t * T1
            pltpu.sync_copy(ids_hbm.at[pl.ds(base, T1)], ids_v)
            pltpu.sync_copy(feat_hbm.at[pl.ds(base, T1), :], feat_v)

            @pl.loop(0, T1 // NLANE)
            def _(g):
                gbase = base + g * NLANE

                @pl.when((gbase < rhi) & (gbase + NLANE > rlo))
                def _():
                    idvec = ids_v[pl.ds(g * NLANE, NLANE)]
                    m0 = idvec[0]
                    full = ((gbase >= rlo) & (gbase + NLANE <= rhi)
                            & (m0 == idvec[NLANE - 1]))

                    @pl.when(full)
                    def _():
                        cross(m0)
                        ls = m0 - st_s[0] * WSEG
                        row = g * NLANE
                        for j in range(D // NLANE):
                            sl = pl.ds(j * NLANE, NLANE)
                            mx = accm_v[ls, sl]
                            sm = accs_v[ls, sl]
                            for r in range(NLANE):
                                v = feat_v[row + r, sl]
                                mx = jnp.maximum(mx, v)
                                sm = sm + v
                            accm_v[ls, sl] = mx
                            accs_v[ls, sl] = sm
                        cnt_s[ls] = cnt_s[ls] + jnp.float32(NLANE)

                    @pl.when(jnp.logical_not(full))
                    def _():
                        for r in range(NLANE):
                            i = gbase + r

                            @pl.when((i >= rlo) & (i < rhi))
                            def _(r=r, i=i):
                                m = idvec[r]
                                cross(m)
                                ls = m - st_s[0] * WSEG
                                for j in range(D // NLANE):
                                    sl = pl.ds(j * NLANE, NLANE)
                                    v = feat_v[g * NLANE + r, sl]
                                    accm_v[ls, sl] = jnp.maximum(
                                        accm_v[ls, sl], v)
                                    accs_v[ls, sl] = accs_v[ls, sl] + v
                                cnt_s[ls] = cnt_s[ls] + jnp.float32(1.0)

        fc = st_s[0]
        flush(fc)
        reinit()

        @pl.loop(fc + 1, w1)
        def _(w):
            flush(w)


def _bn_gelu_body(sum_ref, max_ref, cnt_ref, gamma_ref, beta_ref, out_ref):
    c = cnt_ref[...]
    nonempty = c > 0.0
    s = jnp.where(nonempty,
                  jnp.where(nonempty, max_ref[...], 0.0)
                  + sum_ref[...] / jnp.maximum(c, 1.0),
                  0.0)
    inv_m = jnp.float32(1.0 / M)
    mu = jnp.sum(s, axis=0, keepdims=True) * inv_m
    valid = lax.broadcasted_iota(jnp.int32, (M_PAD, D), 0) < M
    d = jnp.where(valid, s - mu, 0.0)
    var = jnp.sum(d * d, axis=0, keepdims=True) * inv_m
    x = (s - mu) * lax.rsqrt(var + 1e-5) * gamma_ref[...] + beta_ref[...]
    out_ref[...] = 0.5 * x * (1.0 + lax.erf(x * jnp.float32(0.7071067811865476)))


@functools.partial(
    pl.kernel,
    out_type=jax.ShapeDtypeStruct((N, D), jnp.float32),
    mesh=_MESH,
    scratch_types=[
        pltpu.VMEM((G // 128, 128), jnp.int32),
        pltpu.VMEM((G, D), jnp.float32),
        pltpu.VMEM((G, D), jnp.float32),
    ],
)
def _gather_add(feat_hbm, ids2_hbm, table_hbm, out_hbm, idx_v, feat_v, rows_v):
    wid = _worker_id()
    ntiles = N // G

    @pl.loop(wid, ntiles, step=NW)
    def _(t):
        base = t * G
        pltpu.sync_copy(ids2_hbm.at[pl.ds(t * (G // 128), G // 128), :], idx_v)
        pltpu.sync_copy(feat_hbm.at[pl.ds(base, G), :], feat_v)
        for j in range(G // 128):
            pltpu.sync_copy(table_hbm.at[idx_v.at[j]],
                            rows_v.at[pl.ds(j * 128, 128), :])

        @pl.loop(0, G)
        def _(i):
            for j in range(D // NLANE):
                sl = pl.ds(j * NLANE, NLANE)
                rows_v[i, sl] = rows_v[i, sl] + feat_v[i, sl]

        pltpu.sync_copy(rows_v, out_hbm.at[pl.ds(base, G), :])


def kernel(feat, segment_ids, bn_gamma, bn_beta):
    ids = segment_ids.astype(jnp.int32)

    win_row_start = jnp.searchsorted(
        ids, jnp.arange(NWIN + 1, dtype=jnp.int32) * WSEG).astype(jnp.int32)
    target = jnp.arange(NW + 1, dtype=jnp.int32) * (N // NW)
    wk = jnp.searchsorted(win_row_start, target, side="left").astype(jnp.int32)
    wk = wk.at[0].set(0).at[-1].set(NWIN)
    rows = win_row_start[wk]
    params = jnp.concatenate(
        [jnp.stack([wk[:-1], wk[1:], rows[:-1], rows[1:]], axis=1),
         jnp.zeros((NW, NLANE - 4), jnp.int32)], axis=1)

    seg_sum, seg_max, seg_cnt = _seg_reduce(feat, ids, params)

    s_act = pl.pallas_call(
        _bn_gelu_body,
        out_shape=jax.ShapeDtypeStruct((M_PAD, D), jnp.float32),
    )(seg_sum, seg_max, seg_cnt,
      bn_gamma.reshape(1, D).astype(jnp.float32),
      bn_beta.reshape(1, D).astype(jnp.float32))

    ids2 = ids.reshape(N // 128, 128)
    return _gather_add(feat, ids2, s_act)


# speedup vs baseline: 2.8605x; 2.1792x over previous
import functools

import jax
import jax.numpy as jnp
from jax import lax
from jax.experimental import pallas as pl
from jax.experimental.pallas import tpu as pltpu
from jax.experimental.pallas import tpu_sc as plsc

N = 320000
D = 128
M = 10000

NW = 32
WSEG = 32
NWIN = -(-M // WSEG)
M_PAD = NWIN * WSEG
T1 = 320
G = 128
NLANE = 16
NEG = float("-inf")

_MESH = plsc.VectorSubcoreMesh(core_axis_name="c", subcore_axis_name="s")


def _worker_id():
    return lax.axis_index("s") * 2 + lax.axis_index("c")


@functools.partial(
    pl.kernel,
    out_type=(
        jax.ShapeDtypeStruct((M_PAD, D), jnp.float32),
        jax.ShapeDtypeStruct((M_PAD, D), jnp.float32),
        jax.ShapeDtypeStruct((M_PAD, D), jnp.float32),
    ),
    mesh=_MESH,
    scratch_types=[
        pltpu.VMEM((T1, D), jnp.float32),
        pltpu.VMEM((T1, D), jnp.float32),
        pltpu.VMEM((T1 + NLANE,), jnp.int32),
        pltpu.VMEM((T1 + NLANE,), jnp.int32),
        pltpu.VMEM((WSEG, D), jnp.float32),
        pltpu.VMEM((WSEG, D), jnp.float32),
        pltpu.VMEM((WSEG, D), jnp.float32),
        pltpu.VMEM((NLANE,), jnp.int32),
        pltpu.SMEM((2,), jnp.int32),
        pltpu.SMEM((WSEG,), jnp.float32),
        pltpu.SemaphoreType.DMA,
        pltpu.SemaphoreType.DMA,
    ],
)
def _seg_reduce(feat_hbm, ids_hbm, params_hbm, sum_hbm, max_hbm, cnt_hbm,
                feat_a, feat_b, ids_a, ids_b, accs_v, accm_v, cstg_v, par_v,
                st_s, cnt_s, sem_a, sem_b):
    wid = _worker_id()
    pltpu.sync_copy(params_hbm.at[wid], par_v)
    pv = par_v[...]
    w0 = pv[0]
    w1 = pv[1]
    rlo = pv[2]
    rhi = pv[3]

    bufs = ((feat_a, ids_a, sem_a), (feat_b, ids_b, sem_b))

    def issue_in(t, b):
        fv, iv, sem = bufs[b]
        base = t * T1
        pltpu.async_copy(ids_hbm.at[pl.ds(base, T1)], iv.at[pl.ds(0, T1)],
                         sem)
        pltpu.async_copy(feat_hbm.at[pl.ds(base, T1), :], fv, sem)

    def wait_in(b):
        fv, iv, sem = bufs[b]
        pltpu.make_async_copy(ids_hbm.at[pl.ds(0, T1)], iv.at[pl.ds(0, T1)],
                              sem).wait()
        pltpu.make_async_copy(feat_hbm.at[pl.ds(0, T1), :], fv, sem).wait()

    def reinit():
        zero = jnp.zeros((NLANE,), jnp.float32)
        neg = jnp.full((NLANE,), NEG, jnp.float32)

        @pl.loop(0, WSEG)
        def _(s):
            for j in range(D // NLANE):
                accs_v[s, pl.ds(j * NLANE, NLANE)] = zero
                accm_v[s, pl.ds(j * NLANE, NLANE)] = neg
            cnt_s[s] = jnp.float32(0.0)

    def flush(w):
        @pl.loop(0, WSEG)
        def _(s):
            cvec = jnp.full((NLANE,), cnt_s[s])
            for j in range(D // NLANE):
                cstg_v[s, pl.ds(j * NLANE, NLANE)] = cvec

        base = w * WSEG
        pltpu.sync_copy(accs_v, sum_hbm.at[pl.ds(base, WSEG), :])
        pltpu.sync_copy(accm_v, max_hbm.at[pl.ds(base, WSEG), :])
        pltpu.sync_copy(cstg_v, cnt_hbm.at[pl.ds(base, WSEG), :])

    def cross(m):
        wt = m // WSEG
        cw = st_s[0]

        @pl.when(wt > cw)
        def _():
            flush(cw)
            reinit()

            @pl.loop(cw + 1, wt)
            def _(w):
                flush(w)

            st_s[0] = wt

    def accum_group_one_seg(fv, ls, row):
        for j in range(D // NLANE):
            sl = pl.ds(j * NLANE, NLANE)
            mx = accm_v[ls, sl]
            sm = accs_v[ls, sl]
            for r in range(NLANE):
                v = fv[row + r, sl]
                mx = jnp.maximum(mx, v)
                sm = sm + v
            accm_v[ls, sl] = mx
            accs_v[ls, sl] = sm
        cnt_s[ls] = cnt_s[ls] + jnp.float32(NLANE)

    def accum_row(fv, ls, row):
        for j in range(D // NLANE):
            sl = pl.ds(j * NLANE, NLANE)
            v = fv[row, sl]
            accm_v[ls, sl] = jnp.maximum(accm_v[ls, sl], v)
            accs_v[ls, sl] = accs_v[ls, sl] + v
        cnt_s[ls] = cnt_s[ls] + jnp.float32(1.0)

    def fast_tile(b):
        fv, iv, _ = bufs[b]
        wb = st_s[0] * WSEG

        @pl.loop(0, T1 // NLANE)
        def _(g):
            idvec = iv[pl.ds(g * NLANE, NLANE)]
            m0 = idvec[0]
            one_seg = m0 == idvec[NLANE - 1]

            @pl.when(one_seg)
            def _():
                accum_group_one_seg(fv, m0 - wb, g * NLANE)

            @pl.when(jnp.logical_not(one_seg))
            def _():
                @pl.loop(g * NLANE, g * NLANE + NLANE)
                def _(i):
                    m = iv[pl.ds(i, NLANE)][0]
                    accum_row(fv, m - wb, i)

    def careful_tile(t, b):
        fv, iv, _ = bufs[b]
        base = t * T1

        @pl.loop(0, T1 // NLANE)
        def _(g):
            gbase = base + g * NLANE

            @pl.when((gbase < rhi) & (gbase + NLANE > rlo))
            def _(g=g, gbase=gbase):
                idvec = iv[pl.ds(g * NLANE, NLANE)]
                m0 = idvec[0]
                full = ((gbase >= rlo) & (gbase + NLANE <= rhi)
                        & (m0 == idvec[NLANE - 1]))

                @pl.when(full)
                def _():
                    cross(m0)
                    accum_group_one_seg(fv, m0 - st_s[0] * WSEG, g * NLANE)

                @pl.when(jnp.logical_not(full))
                def _():
                    ilo = jnp.maximum(gbase, rlo) - base
                    ihi = jnp.minimum(gbase + NLANE, rhi) - base

                    @pl.loop(ilo, ihi)
                    def _(i):
                        m = iv[pl.ds(i, NLANE)][0]
                        cross(m)
                        accum_row(fv, m - st_s[0] * WSEG, i)

    @pl.when(w1 > w0)
    def _():
        reinit()
        st_s[0] = w0

        t0 = rlo // T1
        t1 = (rhi + T1 - 1) // T1
        nt = t1 - t0

        @pl.when(nt > 0)
        def _():
            issue_in(t0, 0)

        @pl.loop(0, nt, step=2)
        def _(k):
            for b in range(2):
                kk = k + b

                @pl.when(kk < nt)
                def _(kk=kk, b=b):
                    t = t0 + kk
                    wait_in(b)

                    @pl.when(kk + 1 < nt)
                    def _():
                        issue_in(t + 1, 1 - b)

                    base = t * T1
                    iv = bufs[b][1]
                    idlast = iv[pl.ds(T1 - NLANE, NLANE)][NLANE - 1]
                    tile_fast = ((base >= rlo) & (base + T1 <= rhi)
                                 & (idlast // WSEG == st_s[0]))

                    @pl.when(tile_fast)
                    def _():
                        fast_tile(b)

                    @pl.when(jnp.logical_not(tile_fast))
                    def _():
                        careful_tile(t, b)

        fc = st_s[0]
        flush(fc)
        reinit()

        @pl.loop(fc + 1, w1)
        def _(w):
            flush(w)


def _bn_gelu_body(sum_ref, max_ref, cnt_ref, gamma_ref, beta_ref, out_ref):
    c = cnt_ref[...]
    nonempty = c > 0.0
    s = jnp.where(nonempty,
                  jnp.where(nonempty, max_ref[...], 0.0)
                  + sum_ref[...] / jnp.maximum(c, 1.0),
                  0.0)
    inv_m = jnp.float32(1.0 / M)
    mu = jnp.sum(s, axis=0, keepdims=True) * inv_m
    valid = lax.broadcasted_iota(jnp.int32, (M_PAD, D), 0) < M
    d = jnp.where(valid, s - mu, 0.0)
    var = jnp.sum(d * d, axis=0, keepdims=True) * inv_m
    x = (s - mu) * lax.rsqrt(var + 1e-5) * gamma_ref[...] + beta_ref[...]
    out_ref[...] = 0.5 * x * (1.0 + lax.erf(x * jnp.float32(0.7071067811865476)))


@functools.partial(
    pl.kernel,
    out_type=jax.ShapeDtypeStruct((N, D), jnp.float32),
    mesh=_MESH,
    scratch_types=[
        pltpu.VMEM((1, 128), jnp.int32),
        pltpu.VMEM((1, 128), jnp.int32),
        pltpu.VMEM((G, D), jnp.float32),
        pltpu.VMEM((G, D), jnp.float32),
        pltpu.VMEM((G, D), jnp.float32),
        pltpu.VMEM((G, D), jnp.float32),
        pltpu.SemaphoreType.DMA,
        pltpu.SemaphoreType.DMA,
        pltpu.SemaphoreType.DMA,
        pltpu.SemaphoreType.DMA,
        pltpu.SemaphoreType.DMA,
        pltpu.SemaphoreType.DMA,
    ],
)
def _gather_add(feat_hbm, ids2_hbm, table_hbm, out_hbm,
                idx_a, idx_b, feat_a, feat_b, rows_a, rows_b,
                isem_a, isem_b, gsem_a, gsem_b, osem_a, osem_b):
    wid = _worker_id()
    ntiles = N // G
    nmy = (ntiles - wid + NW - 1) // NW

    bufs = ((idx_a, feat_a, rows_a, isem_a, gsem_a, osem_a),
            (idx_b, feat_b, rows_b, isem_b, gsem_b, osem_b))

    def tile_of(k):
        return wid + k * NW

    def issue_in(k, b):
        idx_v, feat_v, _, isem, _, _ = bufs[b]
        t = tile_of(k)
        pltpu.async_copy(ids2_hbm.at[pl.ds(t, 1), :], idx_v, isem)
        pltpu.async_copy(feat_hbm.at[pl.ds(t * G, G), :], feat_v, isem)

    def wait_in(b):
        idx_v, feat_v, _, isem, _, _ = bufs[b]
        pltpu.make_async_copy(ids2_hbm.at[pl.ds(0, 1), :], idx_v, isem).wait()
        pltpu.make_async_copy(feat_hbm.at[pl.ds(0, G), :], feat_v, isem).wait()

    def issue_gather(b):
        idx_v, _, rows_v, _, gsem, _ = bufs[b]
        pltpu.async_copy(table_hbm.at[idx_v.at[0]], rows_v, gsem)

    def wait_gather(b):
        idx_v, _, rows_v, _, gsem, _ = bufs[b]
        pltpu.make_async_copy(table_hbm.at[idx_v.at[0]], rows_v, gsem).wait()

    def issue_out(k, b):
        _, _, rows_v, _, _, osem = bufs[b]
        pltpu.async_copy(rows_v, out_hbm.at[pl.ds(tile_of(k) * G, G), :], osem)

    def wait_out(b):
        _, _, rows_v, _, _, osem = bufs[b]
        pltpu.make_async_copy(rows_v, out_hbm.at[pl.ds(0, G), :], osem).wait()

    @pl.when(nmy > 0)
    def _():
        issue_in(0, 0)
        wait_in(0)
        issue_gather(0)

    @pl.loop(0, nmy, step=2)
    def _(k):
        for b in range(2):
            kk = k + b

            @pl.when(kk < nmy)
            def _(kk=kk, b=b):
                _, feat_v, rows_v, _, _, _ = bufs[b]

                @pl.when(kk + 1 < nmy)
                def _():
                    issue_in(kk + 1, 1 - b)

                wait_gather(b)

                @pl.loop(0, G)
                def _(i):
                    for j in range(D // NLANE):
                        sl = pl.ds(j * NLANE, NLANE)
                        rows_v[i, sl] = rows_v[i, sl] + feat_v[i, sl]

                issue_out(kk, b)

                @pl.when(kk + 1 < nmy)
                def _():
                    wait_in(1 - b)

                    @pl.when(kk >= 1)
                    def _():
                        wait_out(1 - b)

                    issue_gather(1 - b)

    for b in range(2):
        @pl.when(nmy > b)
        def _(b=b):
            wait_out(b)


def kernel(feat, segment_ids, bn_gamma, bn_beta):
    ids = segment_ids.astype(jnp.int32)

    win_row_start = jnp.searchsorted(
        ids, jnp.arange(NWIN + 1, dtype=jnp.int32) * WSEG).astype(jnp.int32)
    target = jnp.arange(NW + 1, dtype=jnp.int32) * (N // NW)
    wk = jnp.searchsorted(win_row_start, target, side="left").astype(jnp.int32)
    wk = wk.at[0].set(0).at[-1].set(NWIN)
    rows = win_row_start[wk]
    params = jnp.concatenate(
        [jnp.stack([wk[:-1], wk[1:], rows[:-1], rows[1:]], axis=1),
         jnp.zeros((NW, NLANE - 4), jnp.int32)], axis=1)

    seg_sum, seg_max, seg_cnt = _seg_reduce(feat, ids, params)

    s_act = pl.pallas_call(
        _bn_gelu_body,
        out_shape=jax.ShapeDtypeStruct((M_PAD, D), jnp.float32),
    )(seg_sum, seg_max, seg_cnt,
      bn_gamma.reshape(1, D).astype(jnp.float32),
      bn_beta.reshape(1, D).astype(jnp.float32))

    ids2 = ids.reshape(N // 128, 128)
    return _gather_add(feat, ids2, s_act)


# speedup vs baseline: 2.9917x; 1.0459x over previous
import functools

import jax
import jax.numpy as jnp
from jax import lax
from jax.experimental import pallas as pl
from jax.experimental.pallas import tpu as pltpu
from jax.experimental.pallas import tpu_sc as plsc

N = 320000
D = 128
M = 10000

NW = 32
WSEG = 32
NWIN = -(-M // WSEG)
M_PAD = NWIN * WSEG
T1 = 320
G = 128
NLANE = 16
NEG = float("-inf")

_MESH = plsc.VectorSubcoreMesh(core_axis_name="c", subcore_axis_name="s")


def _worker_id():
    return lax.axis_index("s") * 2 + lax.axis_index("c")


@functools.partial(
    pl.kernel,
    out_type=(
        jax.ShapeDtypeStruct((M_PAD, D), jnp.float32),
        jax.ShapeDtypeStruct((M_PAD, D), jnp.float32),
        jax.ShapeDtypeStruct((M_PAD, D), jnp.float32),
    ),
    mesh=_MESH,
    scratch_types=[
        pltpu.VMEM((T1, D), jnp.float32),
        pltpu.VMEM((T1, D), jnp.float32),
        pltpu.VMEM((T1 + NLANE,), jnp.int32),
        pltpu.VMEM((T1 + NLANE,), jnp.int32),
        pltpu.VMEM((WSEG, D), jnp.float32),
        pltpu.VMEM((WSEG, D), jnp.float32),
        pltpu.VMEM((WSEG, D), jnp.float32),
        pltpu.VMEM((NLANE,), jnp.int32),
        pltpu.SMEM((2,), jnp.int32),
        pltpu.SMEM((WSEG,), jnp.float32),
        pltpu.SemaphoreType.DMA,
        pltpu.SemaphoreType.DMA,
    ],
)
def _seg_reduce(feat_hbm, ids_hbm, params_hbm, sum_hbm, max_hbm, cnt_hbm,
                feat_a, feat_b, ids_a, ids_b, accs_v, accm_v, cstg_v, par_v,
                st_s, cnt_s, sem_a, sem_b):
    wid = _worker_id()
    pltpu.sync_copy(params_hbm.at[wid], par_v)
    pv = par_v[...]
    w0 = pv[0]
    w1 = pv[1]
    rlo = pv[2]
    rhi = pv[3]

    bufs = ((feat_a, ids_a, sem_a), (feat_b, ids_b, sem_b))

    def issue_in(t, b):
        fv, iv, sem = bufs[b]
        base = t * T1
        pltpu.async_copy(ids_hbm.at[pl.ds(base, T1)], iv.at[pl.ds(0, T1)],
                         sem)
        pltpu.async_copy(feat_hbm.at[pl.ds(base, T1), :], fv, sem)

    def wait_in(b):
        fv, iv, sem = bufs[b]
        pltpu.make_async_copy(ids_hbm.at[pl.ds(0, T1)], iv.at[pl.ds(0, T1)],
                              sem).wait()
        pltpu.make_async_copy(feat_hbm.at[pl.ds(0, T1), :], fv, sem).wait()

    def reinit():
        zero = jnp.zeros((NLANE,), jnp.float32)
        neg = jnp.full((NLANE,), NEG, jnp.float32)

        @pl.loop(0, WSEG)
        def _(s):
            for j in range(D // NLANE):
                accs_v[s, pl.ds(j * NLANE, NLANE)] = zero
                accm_v[s, pl.ds(j * NLANE, NLANE)] = neg
            cnt_s[s] = jnp.float32(0.0)

    def flush(w):
        @pl.loop(0, WSEG)
        def _(s):
            cvec = jnp.full((NLANE,), cnt_s[s])
            for j in range(D // NLANE):
                cstg_v[s, pl.ds(j * NLANE, NLANE)] = cvec

        base = w * WSEG
        pltpu.sync_copy(accs_v, sum_hbm.at[pl.ds(base, WSEG), :])
        pltpu.sync_copy(accm_v, max_hbm.at[pl.ds(base, WSEG), :])
        pltpu.sync_copy(cstg_v, cnt_hbm.at[pl.ds(base, WSEG), :])

    def cross(m):
        wt = m // WSEG
        cw = st_s[0]

        @pl.when(wt > cw)
        def _():
            flush(cw)
            reinit()

            @pl.loop(cw + 1, wt)
            def _(w):
                flush(w)

            st_s[0] = wt

    def accum_group_one_seg(fv, ls, row):
        for j in range(D // NLANE):
            sl = pl.ds(j * NLANE, NLANE)
            mx = accm_v[ls, sl]
            sm = accs_v[ls, sl]
            for r in range(NLANE):
                v = fv[row + r, sl]
                mx = jnp.maximum(mx, v)
                sm = sm + v
            accm_v[ls, sl] = mx
            accs_v[ls, sl] = sm
        cnt_s[ls] = cnt_s[ls] + jnp.float32(NLANE)

    def accum_row(fv, ls, row):
        for j in range(D // NLANE):
            sl = pl.ds(j * NLANE, NLANE)
            v = fv[row, sl]
            accm_v[ls, sl] = jnp.maximum(accm_v[ls, sl], v)
            accs_v[ls, sl] = accs_v[ls, sl] + v
        cnt_s[ls] = cnt_s[ls] + jnp.float32(1.0)

    def fast_tile(b):
        fv, iv, _ = bufs[b]
        wb = st_s[0] * WSEG

        @pl.loop(0, T1 // NLANE)
        def _(g):
            idvec = iv[pl.ds(g * NLANE, NLANE)]
            m0 = idvec[0]
            one_seg = m0 == idvec[NLANE - 1]

            @pl.when(one_seg)
            def _():
                accum_group_one_seg(fv, m0 - wb, g * NLANE)

            @pl.when(jnp.logical_not(one_seg))
            def _():
                @pl.loop(g * NLANE, g * NLANE + NLANE)
                def _(i):
                    m = iv[pl.ds(i, NLANE)][0]
                    accum_row(fv, m - wb, i)

    def careful_tile(t, b):
        fv, iv, _ = bufs[b]
        base = t * T1

        @pl.loop(0, T1 // NLANE)
        def _(g):
            gbase = base + g * NLANE

            @pl.when((gbase < rhi) & (gbase + NLANE > rlo))
            def _(g=g, gbase=gbase):
                idvec = iv[pl.ds(g * NLANE, NLANE)]
                m0 = idvec[0]
                full = ((gbase >= rlo) & (gbase + NLANE <= rhi)
                        & (m0 == idvec[NLANE - 1]))

                @pl.when(full)
                def _():
                    cross(m0)
                    accum_group_one_seg(fv, m0 - st_s[0] * WSEG, g * NLANE)

                @pl.when(jnp.logical_not(full))
                def _():
                    ilo = jnp.maximum(gbase, rlo) - base
                    ihi = jnp.minimum(gbase + NLANE, rhi) - base

                    @pl.loop(ilo, ihi)
                    def _(i):
                        m = iv[pl.ds(i, NLANE)][0]
                        cross(m)
                        accum_row(fv, m - st_s[0] * WSEG, i)

    @pl.when(w1 > w0)
    def _():
        reinit()
        st_s[0] = w0

        t0 = rlo // T1
        t1 = (rhi + T1 - 1) // T1
        nt = t1 - t0

        @pl.when(nt > 0)
        def _():
            issue_in(t0, 0)

        @pl.loop(0, nt, step=2)
        def _(k):
            for b in range(2):
                kk = k + b

                @pl.when(kk < nt)
                def _(kk=kk, b=b):
                    t = t0 + kk
                    wait_in(b)

                    @pl.when(kk + 1 < nt)
                    def _():
                        issue_in(t + 1, 1 - b)

                    base = t * T1
                    iv = bufs[b][1]
                    idlast = iv[pl.ds(T1 - NLANE, NLANE)][NLANE - 1]
                    tile_fast = ((base >= rlo) & (base + T1 <= rhi)
                                 & (idlast // WSEG == st_s[0]))

                    @pl.when(tile_fast)
                    def _():
                        fast_tile(b)

                    @pl.when(jnp.logical_not(tile_fast))
                    def _():
                        careful_tile(t, b)

        fc = st_s[0]
        flush(fc)
        reinit()

        @pl.loop(fc + 1, w1)
        def _(w):
            flush(w)


def _bn_gelu_body(sum_ref, max_ref, cnt_ref, gamma_ref, beta_ref, out_ref):
    c = cnt_ref[...]
    nonempty = c > 0.0
    s = jnp.where(nonempty,
                  jnp.where(nonempty, max_ref[...], 0.0)
                  + sum_ref[...] / jnp.maximum(c, 1.0),
                  0.0)
    inv_m = jnp.float32(1.0 / M)
    mu = jnp.sum(s, axis=0, keepdims=True) * inv_m
    valid = lax.broadcasted_iota(jnp.int32, (M_PAD, D), 0) < M
    d = jnp.where(valid, s - mu, 0.0)
    var = jnp.sum(d * d, axis=0, keepdims=True) * inv_m
    x = (s - mu) * lax.rsqrt(var + 1e-5) * gamma_ref[...] + beta_ref[...]
    out_ref[...] = 0.5 * x * (1.0 + lax.erf(x * jnp.float32(0.7071067811865476)))


@functools.partial(
    pl.kernel,
    out_type=jax.ShapeDtypeStruct((N, D), jnp.float32),
    mesh=_MESH,
    scratch_types=[
        pltpu.VMEM((1, 128), jnp.int32),
        pltpu.VMEM((1, 128), jnp.int32),
        pltpu.VMEM((G, D), jnp.float32),
        pltpu.VMEM((G, D), jnp.float32),
        pltpu.VMEM((G, D), jnp.float32),
        pltpu.VMEM((G, D), jnp.float32),
        pltpu.SemaphoreType.DMA,
        pltpu.SemaphoreType.DMA,
        pltpu.SemaphoreType.DMA,
        pltpu.SemaphoreType.DMA,
        pltpu.SemaphoreType.DMA,
        pltpu.SemaphoreType.DMA,
    ],
)
def _gather_add(feat_hbm, ids2_hbm, table_hbm, out_hbm,
                idx_a, idx_b, feat_a, feat_b, rows_a, rows_b,
                isem_a, isem_b, gsem_a, gsem_b, osem_a, osem_b):
    wid = _worker_id()
    ntiles = N // G
    nmy = (ntiles - wid + NW - 1) // NW

    bufs = ((idx_a, feat_a, rows_a, isem_a, gsem_a, osem_a),
            (idx_b, feat_b, rows_b, isem_b, gsem_b, osem_b))

    def tile_of(k):
        return wid + k * NW

    def issue_in(k, b):
        idx_v, feat_v, _, isem, _, _ = bufs[b]
        t = tile_of(k)
        pltpu.async_copy(ids2_hbm.at[pl.ds(t, 1), :], idx_v, isem)
        pltpu.async_copy(feat_hbm.at[pl.ds(t * G, G), :], feat_v, isem)

    def wait_in(b):
        idx_v, feat_v, _, isem, _, _ = bufs[b]
        pltpu.make_async_copy(ids2_hbm.at[pl.ds(0, 1), :], idx_v, isem).wait()
        pltpu.make_async_copy(feat_hbm.at[pl.ds(0, G), :], feat_v, isem).wait()

    def issue_gather(b):
        idx_v, _, rows_v, _, gsem, _ = bufs[b]
        pltpu.async_copy(table_hbm.at[idx_v.at[0]], rows_v, gsem)

    def wait_gather(b):
        idx_v, _, rows_v, _, gsem, _ = bufs[b]
        pltpu.make_async_copy(table_hbm.at[idx_v.at[0]], rows_v, gsem).wait()

    def issue_out(k, b):
        _, _, rows_v, _, _, osem = bufs[b]
        pltpu.async_copy(rows_v, out_hbm.at[pl.ds(tile_of(k) * G, G), :], osem)

    def wait_out(b):
        _, _, rows_v, _, _, osem = bufs[b]
        pltpu.make_async_copy(rows_v, out_hbm.at[pl.ds(0, G), :], osem).wait()

    @pl.when(nmy > 0)
    def _():
        issue_in(0, 0)

        @pl.when(nmy > 1)
        def _():
            issue_in(1, 1)

        wait_in(0)
        issue_gather(0)

    @pl.loop(0, nmy, step=2)
    def _(k):
        for b in range(2):
            kk = k + b

            @pl.when(kk < nmy)
            def _(kk=kk, b=b):
                _, feat_v, rows_v, _, _, _ = bufs[b]

                @pl.when(kk + 1 < nmy)
                def _():
                    wait_in(1 - b)

                    @pl.when(kk >= 1)
                    def _():
                        wait_out(1 - b)

                    issue_gather(1 - b)

                wait_gather(b)

                @pl.loop(0, G, step=4)
                def _(i):
                    for r in range(4):
                        for j in range(D // NLANE):
                            sl = pl.ds(j * NLANE, NLANE)
                            rows_v[i + r, sl] = (rows_v[i + r, sl]
                                                 + feat_v[i + r, sl])

                issue_out(kk, b)

                @pl.when(kk + 2 < nmy)
                def _():
                    issue_in(kk + 2, b)

    for b in range(2):
        @pl.when(nmy > b)
        def _(b=b):
            wait_out(b)


def kernel(feat, segment_ids, bn_gamma, bn_beta):
    ids = segment_ids.astype(jnp.int32)

    win_row_start = jnp.searchsorted(
        ids, jnp.arange(NWIN + 1, dtype=jnp.int32) * WSEG).astype(jnp.int32)
    target = jnp.arange(NW + 1, dtype=jnp.int32) * (N // NW)
    wk = jnp.searchsorted(win_row_start, target, side="left").astype(jnp.int32)
    wk = wk.at[0].set(0).at[-1].set(NWIN)
    rows = win_row_start[wk]
    params = jnp.concatenate(
        [jnp.stack([wk[:-1], wk[1:], rows[:-1], rows[1:]], axis=1),
         jnp.zeros((NW, NLANE - 4), jnp.int32)], axis=1)

    seg_sum, seg_max, seg_cnt = _seg_reduce(feat, ids, params)

    s_act = pl.pallas_call(
        _bn_gelu_body,
        out_shape=jax.ShapeDtypeStruct((M_PAD, D), jnp.float32),
    )(seg_sum, seg_max, seg_cnt,
      bn_gamma.reshape(1, D).astype(jnp.float32),
      bn_beta.reshape(1, D).astype(jnp.float32))

    ids2 = ids.reshape(N // 128, 128)
    return _gather_add(feat, ids2, s_act)
